# baseline (speedup 1.0000x reference)
import jax
import jax.numpy as jnp
from jax import lax
from jax.experimental import pallas as pl
from jax.experimental.pallas import tpu as pltpu

_B, _N, _NP1, _NP2, _K1, _K2 = 16, 2048, 512, 128, 32, 64
_R1SQ = 0.2 * 0.2
_R2SQ = 0.4 * 0.4
_NC, _CAT = 50, 16
_PAN = 128


def _mm(a, b):
    return lax.dot_general(a, b, (((1,), (0,)), ((), ())),
                           preferred_element_type=jnp.float32)


def _mmT(a, b):
    return lax.dot_general(a, b, (((0,), (0,)), ((), ())),
                           preferred_element_type=jnp.float32)


def _relu(h):
    return jnp.maximum(h, 0.0)


def _fps_loop(px, py, pz, npts, cx_ref, cy_ref, cz_ref):
    b, n = px.shape
    io = lax.broadcasted_iota(jnp.int32, (b, n), 1)
    ioc = lax.broadcasted_iota(jnp.int32, (b, npts), 1)

    def step(t, carry):
        mind, last, cxs, cys, czs = carry
        oh = io == last
        xl = jnp.sum(jnp.where(oh, px, 0.0), axis=1, keepdims=True)
        yl = jnp.sum(jnp.where(oh, py, 0.0), axis=1, keepdims=True)
        zl = jnp.sum(jnp.where(oh, pz, 0.0), axis=1, keepdims=True)
        slot = ioc == t
        cxs = jnp.where(slot, xl, cxs)
        cys = jnp.where(slot, yl, cys)
        czs = jnp.where(slot, zl, czs)
        d = (px - xl) ** 2 + (py - yl) ** 2 + (pz - zl) ** 2
        mind = jnp.minimum(mind, d)
        mx = jnp.max(mind, axis=1, keepdims=True)
        nxt = jnp.min(jnp.where(mind == mx, io, n), axis=1, keepdims=True)
        return mind, nxt, cxs, cys, czs

    mind0 = jnp.full((b, n), 1e10, jnp.float32)
    last0 = jnp.zeros((b, 1), jnp.int32)
    z = jnp.zeros((b, npts), jnp.float32)
    _, _, cxs, cys, czs = lax.fori_loop(0, npts, step,
                                        (mind0, last0, z, z, z))
    cx_ref[...] = cxs
    cy_ref[...] = cys
    cz_ref[...] = czs


def _k1_body(px_ref, py_ref, pz_ref,
             c1x_ref, c1y_ref, c1z_ref, c2x_ref, c2y_ref, c2z_ref):
    _fps_loop(px_ref[...], py_ref[...], pz_ref[...], _NP1,
              c1x_ref, c1y_ref, c1z_ref)
    _fps_loop(c1x_ref[...], c1y_ref[...], c1z_ref[...], _NP2,
              c2x_ref, c2y_ref, c2z_ref)


def _tri():
    r = lax.broadcasted_iota(jnp.int32, (_PAN, _PAN), 0)
    c = lax.broadcasted_iota(jnp.int32, (_PAN, _PAN), 1)
    return (r < c).astype(jnp.float32)


def _ball_rank(pxc, pyc, pzc, cx, cy, cz, rsq, kk, d2_ref, sr_ref):
    n = pxc.shape[0]
    s = cx.shape[1]
    tl = _tri()
    off = jnp.zeros((1, s), jnp.float32)
    cnt = jnp.zeros((s, 1), jnp.float32)
    mrs = []
    ones = jnp.ones((_PAN, 1), jnp.float32)
    for p in range(n // _PAN):
        sl = slice(p * _PAN, (p + 1) * _PAN)
        d2p = ((pxc[sl] - cx) ** 2 + (pyc[sl] - cy) ** 2
               + (pzc[sl] - cz) ** 2)
        mf = (d2p <= rsq).astype(jnp.float32)
        rank = _mmT(tl, mf) + off
        off = off + jnp.sum(mf, axis=0, keepdims=True)
        self_f = mf * (rank < float(kk)).astype(jnp.float32)
        d2_ref[sl, :] = d2p
        srp = jnp.where(self_f > 0, rank, -1.0)
        sr_ref[sl, :] = srp
        mrs.append(jnp.max(srp))
        cnt = cnt + _mmT(self_f, ones)
    return cnt, mrs


def _interp_w(d2p):
    m1 = jnp.min(d2p, axis=1, keepdims=True)
    d2b = jnp.where(d2p == m1, 1e30, d2p)
    m2 = jnp.min(d2b, axis=1, keepdims=True)
    d2c = jnp.where(d2b == m2, 1e30, d2b)
    m3 = jnp.min(d2c, axis=1, keepdims=True)
    self_f = (d2p <= m3).astype(jnp.float32)
    w = self_f / jnp.maximum(d2p, 1e-10)
    return w / jnp.sum(w, axis=1, keepdims=True)


def _k2_body(pxc_ref, pyc_ref, pzc_ref, x6_ref, cat_ref,
             c1x_ref, c1y_ref, c1z_ref, l1x_ref, l1y_ref, l1z_ref,
             c2x_ref, c2y_ref, c2z_ref, m2x_ref, m2y_ref, m2z_ref,
             s1w0_ref, s1b0_ref, s1w1_ref, s1b1_ref, s1w2_ref, s1b2_ref,
             s2w0_ref, s2b0_ref, s2w1_ref, s2b1_ref, s2w2_ref, s2b2_ref,
             s3w0_ref, s3b0_ref, s3w1_ref, s3b1_ref, s3w2_ref, s3b2_ref,
             f3w0_ref, f3b0_ref, f3w1_ref, f3b1_ref,
             f2w0_ref, f2b0_ref, f2w1_ref, f2b1_ref,
             f1w0_ref, f1b0_ref, f1w1_ref, f1b1_ref, f1w2_ref, f1b2_ref,
             hw0_ref, hb0_ref, hw1_ref, hb1_ref,
             out_ref,
             d2a_ref, sra_ref, phi_ref, l1f_ref, srb_ref, l1up_ref):
    pxc = pxc_ref[0]
    pyc = pyc_ref[0]
    pzc = pzc_ref[0]
    c1x = c1x_ref[0]
    c1y = c1y_ref[0]
    c1z = c1z_ref[0]

    for p in range(_N // 512):
        sl = slice(p * 512, (p + 1) * 512)
        h = _relu(_mm(x6_ref[0][sl, :], s1w0_ref[...]) + s1b0_ref[...])
        h = _relu(_mm(h, s1w1_ref[...]) + s1b1_ref[...])
        h = _relu(_mm(h, s1w2_ref[...]) + s1b2_ref[...])
        phi_ref[sl, :] = h

    cnt1, mrs1 = _ball_rank(pxc, pyc, pzc, c1x, c1y, c1z, _R1SQ, _K1,
                            d2a_ref, sra_ref)
    t1 = jnp.max(cnt1).astype(jnp.int32)

    def sa1_j(j, out):
        jf = j.astype(jnp.float32)
        acc = jnp.zeros((_NP1, 128), jnp.float32)
        for p in range(_N // _PAN):
            sl = slice(p * _PAN, (p + 1) * _PAN)

            def _panel(sl=sl):
                sp = (sra_ref[sl, :] == jf).astype(jnp.float32)
                return _mmT(sp, phi_ref[sl, :])

            acc = acc + lax.cond(jf <= mrs1[p], _panel,
                                 lambda: jnp.zeros((_NP1, 128), jnp.float32))
        return jnp.where(jf < cnt1, jnp.maximum(out, acc), out)

    l1f = lax.fori_loop(0, t1, sa1_j, jnp.full((_NP1, 128), -1e30,
                                               jnp.float32))
    l1f_ref[...] = l1f

    l1xc = l1x_ref[0]
    l1yc = l1y_ref[0]
    l1zc = l1z_ref[0]
    c2x = c2x_ref[0]
    c2y = c2y_ref[0]
    c2z = c2z_ref[0]
    m2x = m2x_ref[0]
    m2y = m2y_ref[0]
    m2z = m2z_ref[0]

    tl = _tri()
    off = jnp.zeros((1, _NP2), jnp.float32)
    cnt2 = jnp.zeros((_NP2, 1), jnp.float32)
    mrs2 = []
    ones = jnp.ones((_PAN, 1), jnp.float32)
    for p in range(_NP1 // _PAN):
        sl = slice(p * _PAN, (p + 1) * _PAN)
        d2p = ((l1xc[sl] - c2x) ** 2 + (l1yc[sl] - c2y) ** 2
               + (l1zc[sl] - c2z) ** 2)
        mf = (d2p <= _R2SQ).astype(jnp.float32)
        rank = _mmT(tl, mf) + off
        off = off + jnp.sum(mf, axis=0, keepdims=True)
        self_f = mf * (rank < float(_K2)).astype(jnp.float32)
        srp = jnp.where(self_f > 0, rank, -1.0)
        srb_ref[sl, :] = srp
        mrs2.append(jnp.max(srp))
        cnt2 = cnt2 + _mmT(self_f, ones)
    t2 = jnp.max(cnt2).astype(jnp.int32)

    def sa2_j(j, out):
        jf = j.astype(jnp.float32)
        gf = jnp.zeros((_NP2, 128), jnp.float32)
        gx = jnp.zeros((_NP2, 1), jnp.float32)
        gy = jnp.zeros((_NP2, 1), jnp.float32)
        gz = jnp.zeros((_NP2, 1), jnp.float32)
        for p in range(_NP1 // _PAN):
            sl = slice(p * _PAN, (p + 1) * _PAN)

            def _panel(sl=sl):
                sp = (srb_ref[sl, :] == jf).astype(jnp.float32)
                return (_mmT(sp, l1f_ref[sl, :]), _mmT(sp, l1xc[sl]),
                        _mmT(sp, l1yc[sl]), _mmT(sp, l1zc[sl]))

            def _skip():
                return (jnp.zeros((_NP2, 128), jnp.float32),
                        jnp.zeros((_NP2, 1), jnp.float32),
                        jnp.zeros((_NP2, 1), jnp.float32),
                        jnp.zeros((_NP2, 1), jnp.float32))

            df, dx, dy, dz = lax.cond(jf <= mrs2[p], _panel, _skip)
            gf = gf + df
            gx = gx + dx
            gy = gy + dy
            gz = gz + dz
        h = jnp.concatenate([gx - m2x, gy - m2y, gz - m2z, gf], axis=1)
        h = _relu(_mm(h, s2w0_ref[...]) + s2b0_ref[...])
        h = _relu(_mm(h, s2w1_ref[...]) + s2b1_ref[...])
        h = _relu(_mm(h, s2w2_ref[...]) + s2b2_ref[...])
        return jnp.where(jf < cnt2, jnp.maximum(out, h), out)

    l2f = lax.fori_loop(0, t2, sa2_j, jnp.full((_NP2, 256), -1e30,
                                               jnp.float32))

    h3 = jnp.concatenate([m2x, m2y, m2z, l2f], axis=1)
    h3 = _relu(_mm(h3, s3w0_ref[...]) + s3b0_ref[...])
    h3 = _relu(_mm(h3, s3w1_ref[...]) + s3b1_ref[...])
    h3 = _relu(_mm(h3, s3w2_ref[...]) + s3b2_ref[...])
    l3f = jnp.max(h3, axis=0, keepdims=True)

    hf3 = jnp.concatenate([jnp.broadcast_to(l3f, (_NP2, 1024)), l2f], axis=1)
    hf3 = _relu(_mm(hf3, f3w0_ref[...]) + f3b0_ref[...])
    l2up = _relu(_mm(hf3, f3w1_ref[...]) + f3b1_ref[...])

    d2f = ((l1xc - c2x) ** 2 + (l1yc - c2y) ** 2 + (l1zc - c2z) ** 2)
    w2 = _interp_w(d2f)
    interp2 = _mm(w2, l2up)
    hf2 = jnp.concatenate([interp2, l1f_ref[...]], axis=1)
    hf2 = _relu(_mm(hf2, f2w0_ref[...]) + f2b0_ref[...])
    l1up_ref[...] = _relu(_mm(hf2, f2w1_ref[...]) + f2b1_ref[...])

    for p in range(_N // 256):
        sl = slice(p * 256, (p + 1) * 256)
        w1 = _interp_w(d2a_ref[sl, :])
        interp1 = _mm(w1, l1up_ref[...])
        h = jnp.concatenate([interp1, cat_ref[0][sl, :], x6_ref[0][sl, :]],
                            axis=1)
        h = _relu(_mm(h, f1w0_ref[...]) + f1b0_ref[...])
        h = _relu(_mm(h, f1w1_ref[...]) + f1b1_ref[...])
        h = _relu(_mm(h, f1w2_ref[...]) + f1b2_ref[...])
        h = _relu(_mm(h, hw0_ref[...]) + hb0_ref[...])
        out_ref[0, sl, :] = _mm(h, hw1_ref[...]) + hb1_ref[...]


def kernel(x, cat_vec, params):
    xf = x.astype(jnp.float32)
    px = xf[:, :, 0]
    py = xf[:, :, 1]
    pz = xf[:, :, 2]

    c1x, c1y, c1z, c2x, c2y, c2z = pl.pallas_call(
        _k1_body,
        out_shape=[
            jax.ShapeDtypeStruct((_B, _NP1), jnp.float32),
            jax.ShapeDtypeStruct((_B, _NP1), jnp.float32),
            jax.ShapeDtypeStruct((_B, _NP1), jnp.float32),
            jax.ShapeDtypeStruct((_B, _NP2), jnp.float32),
            jax.ShapeDtypeStruct((_B, _NP2), jnp.float32),
            jax.ShapeDtypeStruct((_B, _NP2), jnp.float32),
        ],
    )(px, py, pz)

    catT = jnp.transpose(cat_vec, (0, 2, 1))
    flat = []
    for name in ('sa1', 'sa2', 'sa3', 'fp3', 'fp2', 'fp1', 'head'):
        for (w, b) in params[name]:
            flat.append(w.astype(jnp.float32))
            flat.append(b.astype(jnp.float32).reshape(1, -1))

    def _bs3(a, b_, c):
        return pl.BlockSpec((1, a, b_) if c else (a, b_),
                            (lambda i: (i, 0, 0)) if c else (lambda i: (0, 0)))

    in_specs = (
        [_bs3(_N, 1, True)] * 3
        + [_bs3(_N, 6, True), _bs3(_N, _CAT, True)]
        + [pl.BlockSpec((1, 1, _NP1), lambda i: (i, 0, 0))] * 3
        + [_bs3(_NP1, 1, True)] * 3
        + [pl.BlockSpec((1, 1, _NP2), lambda i: (i, 0, 0))] * 3
        + [_bs3(_NP2, 1, True)] * 3
        + [_bs3(w.shape[0], w.shape[1], False) for w in flat]
    )

    out = pl.pallas_call(
        _k2_body,
        grid=(_B,),
        in_specs=in_specs,
        out_specs=pl.BlockSpec((1, _N, _NC), lambda i: (i, 0, 0)),
        out_shape=jax.ShapeDtypeStruct((_B, _N, _NC), jnp.float32),
        scratch_shapes=[
            pltpu.VMEM((_N, _NP1), jnp.float32),
            pltpu.VMEM((_N, _NP1), jnp.float32),
            pltpu.VMEM((_N, 128), jnp.float32),
            pltpu.VMEM((_NP1, 128), jnp.float32),
            pltpu.VMEM((_NP1, _NP2), jnp.float32),
            pltpu.VMEM((_NP1, 128), jnp.float32),
        ],
    )(px[..., None], py[..., None], pz[..., None], xf, catT,
      c1x[:, None, :], c1y[:, None, :], c1z[:, None, :],
      c1x[..., None], c1y[..., None], c1z[..., None],
      c2x[:, None, :], c2y[:, None, :], c2z[:, None, :],
      c2x[..., None], c2y[..., None], c2z[..., None],
      *flat)
    return jnp.transpose(out, (0, 2, 1))


# speedup vs baseline: 1.3559x; 1.3559x over previous
import jax
import jax.numpy as jnp
from jax import lax
from jax.experimental import pallas as pl
from jax.experimental.pallas import tpu as pltpu

_B, _N, _NP1, _NP2, _K1, _K2 = 16, 2048, 512, 128, 32, 64
_R1SQ = 0.2 * 0.2
_R2SQ = 0.4 * 0.4
_NC, _CAT = 50, 16
_PAN = 128


def _mm(a, b):
    return lax.dot_general(a, b, (((1,), (0,)), ((), ())),
                           preferred_element_type=jnp.float32)


def _mmT(a, b):
    return lax.dot_general(a, b, (((0,), (0,)), ((), ())),
                           preferred_element_type=jnp.float32)


def _relu(h):
    return jnp.maximum(h, 0.0)


def _fps_loop(px, py, pz, npts, cx_ref, cy_ref, cz_ref):
    b, n = px.shape
    io = lax.broadcasted_iota(jnp.int32, (b, n), 1)
    ioc = lax.broadcasted_iota(jnp.int32, (b, npts), 1)

    def step(t, carry):
        mind, last, cxs, cys, czs = carry
        oh = io == last
        xl = jnp.sum(jnp.where(oh, px, 0.0), axis=1, keepdims=True)
        yl = jnp.sum(jnp.where(oh, py, 0.0), axis=1, keepdims=True)
        zl = jnp.sum(jnp.where(oh, pz, 0.0), axis=1, keepdims=True)
        slot = ioc == t
        cxs = jnp.where(slot, xl, cxs)
        cys = jnp.where(slot, yl, cys)
        czs = jnp.where(slot, zl, czs)
        d = (px - xl) ** 2 + (py - yl) ** 2 + (pz - zl) ** 2
        mind = jnp.minimum(mind, d)
        mx = jnp.max(mind, axis=1, keepdims=True)
        nxt = jnp.min(jnp.where(mind == mx, io, n), axis=1, keepdims=True)
        return mind, nxt, cxs, cys, czs

    mind0 = jnp.full((b, n), 1e10, jnp.float32)
    last0 = jnp.zeros((b, 1), jnp.int32)
    z = jnp.zeros((b, npts), jnp.float32)
    _, _, cxs, cys, czs = lax.fori_loop(0, npts, step,
                                        (mind0, last0, z, z, z))
    cx_ref[...] = cxs
    cy_ref[...] = cys
    cz_ref[...] = czs


def _k1_body(px_ref, py_ref, pz_ref,
             c1x_ref, c1y_ref, c1z_ref, c2x_ref, c2y_ref, c2z_ref):
    _fps_loop(px_ref[...], py_ref[...], pz_ref[...], _NP1,
              c1x_ref, c1y_ref, c1z_ref)
    _fps_loop(c1x_ref[...], c1y_ref[...], c1z_ref[...], _NP2,
              c2x_ref, c2y_ref, c2z_ref)


def _tri():
    r = lax.broadcasted_iota(jnp.int32, (_PAN, _PAN), 0)
    c = lax.broadcasted_iota(jnp.int32, (_PAN, _PAN), 1)
    return (r < c).astype(jnp.float32)


def _ball_rank(pxc, pyc, pzc, cx, cy, cz, rsq, kk, d2_ref, sr_ref):
    n = pxc.shape[0]
    s = cx.shape[1]
    tl = _tri()
    off = jnp.zeros((1, s), jnp.float32)
    cnt = jnp.zeros((s, 1), jnp.float32)
    ones = jnp.ones((_PAN, 1), jnp.float32)
    for p in range(n // _PAN):
        sl = slice(p * _PAN, (p + 1) * _PAN)
        d2p = ((pxc[sl] - cx) ** 2 + (pyc[sl] - cy) ** 2
               + (pzc[sl] - cz) ** 2)
        mf = (d2p <= rsq).astype(jnp.float32)
        rank = _mmT(tl, mf) + off
        off = off + jnp.sum(mf, axis=0, keepdims=True)
        self_f = mf * (rank < float(kk)).astype(jnp.float32)
        d2_ref[sl, :] = d2p
        sr_ref[sl, :] = jnp.where(self_f > 0, rank, -1.0)
        cnt = cnt + _mmT(self_f, ones)
    return cnt


def _interp_w(d2p):
    m1 = jnp.min(d2p, axis=1, keepdims=True)
    d2b = jnp.where(d2p == m1, 1e30, d2p)
    m2 = jnp.min(d2b, axis=1, keepdims=True)
    d2c = jnp.where(d2b == m2, 1e30, d2b)
    m3 = jnp.min(d2c, axis=1, keepdims=True)
    self_f = (d2p <= m3).astype(jnp.float32)
    w = self_f / jnp.maximum(d2p, 1e-10)
    return w / jnp.sum(w, axis=1, keepdims=True)


def _k2_body(pxc_ref, pyc_ref, pzc_ref, x6_ref, cat_ref,
             c1x_ref, c1y_ref, c1z_ref, l1x_ref, l1y_ref, l1z_ref,
             c2x_ref, c2y_ref, c2z_ref, m2x_ref, m2y_ref, m2z_ref,
             s1w0_ref, s1b0_ref, s1w1_ref, s1b1_ref, s1w2_ref, s1b2_ref,
             s2w0_ref, s2b0_ref, s2w1_ref, s2b1_ref, s2w2_ref, s2b2_ref,
             s3w0_ref, s3b0_ref, s3w1_ref, s3b1_ref, s3w2_ref, s3b2_ref,
             f3w0_ref, f3b0_ref, f3w1_ref, f3b1_ref,
             f2w0_ref, f2b0_ref, f2w1_ref, f2b1_ref,
             f1w0_ref, f1b0_ref, f1w1_ref, f1b1_ref, f1w2_ref, f1b2_ref,
             hw0_ref, hb0_ref, hw1_ref, hb1_ref,
             out_ref,
             d2a_ref, sra_ref, phi_ref, l1f_ref, srb_ref, l1up_ref):
    pxc = pxc_ref[0]
    pyc = pyc_ref[0]
    pzc = pzc_ref[0]
    c1x = c1x_ref[0]
    c1y = c1y_ref[0]
    c1z = c1z_ref[0]

    for p in range(_N // 512):
        sl = slice(p * 512, (p + 1) * 512)
        h = _relu(_mm(x6_ref[0][sl, :], s1w0_ref[...]) + s1b0_ref[...])
        h = _relu(_mm(h, s1w1_ref[...]) + s1b1_ref[...])
        h = _relu(_mm(h, s1w2_ref[...]) + s1b2_ref[...])
        phi_ref[sl, :] = h

    cnt1 = _ball_rank(pxc, pyc, pzc, c1x, c1y, c1z, _R1SQ, _K1,
                      d2a_ref, sra_ref)
    t1 = jnp.max(cnt1).astype(jnp.int32)

    def sa1_j(j, out):
        jf = j.astype(jnp.float32)
        acc = jnp.zeros((_NP1, 128), jnp.float32)
        for p in range(_N // _PAN):
            sl = slice(p * _PAN, (p + 1) * _PAN)
            sp = (sra_ref[sl, :] == jf).astype(jnp.float32)
            acc = acc + _mmT(sp, phi_ref[sl, :])
        return jnp.where(jf < cnt1, jnp.maximum(out, acc), out)

    l1f = lax.fori_loop(0, t1, sa1_j, jnp.full((_NP1, 128), -1e30,
                                               jnp.float32))
    l1f_ref[...] = l1f

    l1xc = l1x_ref[0]
    l1yc = l1y_ref[0]
    l1zc = l1z_ref[0]
    c2x = c2x_ref[0]
    c2y = c2y_ref[0]
    c2z = c2z_ref[0]
    m2x = m2x_ref[0]
    m2y = m2y_ref[0]
    m2z = m2z_ref[0]

    tl = _tri()
    off = jnp.zeros((1, _NP2), jnp.float32)
    cnt2 = jnp.zeros((_NP2, 1), jnp.float32)
    ones = jnp.ones((_PAN, 1), jnp.float32)
    for p in range(_NP1 // _PAN):
        sl = slice(p * _PAN, (p + 1) * _PAN)
        d2p = ((l1xc[sl] - c2x) ** 2 + (l1yc[sl] - c2y) ** 2
               + (l1zc[sl] - c2z) ** 2)
        mf = (d2p <= _R2SQ).astype(jnp.float32)
        rank = _mmT(tl, mf) + off
        off = off + jnp.sum(mf, axis=0, keepdims=True)
        self_f = mf * (rank < float(_K2)).astype(jnp.float32)
        srb_ref[sl, :] = jnp.where(self_f > 0, rank, -1.0)
        cnt2 = cnt2 + _mmT(self_f, ones)
    t2 = jnp.max(cnt2).astype(jnp.int32)

    def sa2_j(j, out):
        jf = j.astype(jnp.float32)
        gf = jnp.zeros((_NP2, 128), jnp.float32)
        gx = jnp.zeros((_NP2, 1), jnp.float32)
        gy = jnp.zeros((_NP2, 1), jnp.float32)
        gz = jnp.zeros((_NP2, 1), jnp.float32)
        for p in range(_NP1 // _PAN):
            sl = slice(p * _PAN, (p + 1) * _PAN)
            sp = (srb_ref[sl, :] == jf).astype(jnp.float32)
            gf = gf + _mmT(sp, l1f_ref[sl, :])
            gx = gx + _mmT(sp, l1xc[sl])
            gy = gy + _mmT(sp, l1yc[sl])
            gz = gz + _mmT(sp, l1zc[sl])
        h = jnp.concatenate([gx - m2x, gy - m2y, gz - m2z, gf], axis=1)
        h = _relu(_mm(h, s2w0_ref[...]) + s2b0_ref[...])
        h = _relu(_mm(h, s2w1_ref[...]) + s2b1_ref[...])
        h = _relu(_mm(h, s2w2_ref[...]) + s2b2_ref[...])
        return jnp.where(jf < cnt2, jnp.maximum(out, h), out)

    l2f = lax.fori_loop(0, t2, sa2_j, jnp.full((_NP2, 256), -1e30,
                                               jnp.float32))

    h3 = jnp.concatenate([m2x, m2y, m2z, l2f], axis=1)
    h3 = _relu(_mm(h3, s3w0_ref[...]) + s3b0_ref[...])
    h3 = _relu(_mm(h3, s3w1_ref[...]) + s3b1_ref[...])
    h3 = _relu(_mm(h3, s3w2_ref[...]) + s3b2_ref[...])
    l3f = jnp.max(h3, axis=0, keepdims=True)

    hf3 = jnp.concatenate([jnp.broadcast_to(l3f, (_NP2, 1024)), l2f], axis=1)
    hf3 = _relu(_mm(hf3, f3w0_ref[...]) + f3b0_ref[...])
    l2up = _relu(_mm(hf3, f3w1_ref[...]) + f3b1_ref[...])

    d2f = ((l1xc - c2x) ** 2 + (l1yc - c2y) ** 2 + (l1zc - c2z) ** 2)
    w2 = _interp_w(d2f)
    interp2 = _mm(w2, l2up)
    hf2 = jnp.concatenate([interp2, l1f_ref[...]], axis=1)
    hf2 = _relu(_mm(hf2, f2w0_ref[...]) + f2b0_ref[...])
    l1up_ref[...] = _relu(_mm(hf2, f2w1_ref[...]) + f2b1_ref[...])

    for p in range(_N // 256):
        sl = slice(p * 256, (p + 1) * 256)
        w1 = _interp_w(d2a_ref[sl, :])
        interp1 = _mm(w1, l1up_ref[...])
        h = jnp.concatenate([interp1, cat_ref[0][sl, :], x6_ref[0][sl, :]],
                            axis=1)
        h = _relu(_mm(h, f1w0_ref[...]) + f1b0_ref[...])
        h = _relu(_mm(h, f1w1_ref[...]) + f1b1_ref[...])
        h = _relu(_mm(h, f1w2_ref[...]) + f1b2_ref[...])
        h = _relu(_mm(h, hw0_ref[...]) + hb0_ref[...])
        out_ref[0, sl, :] = _mm(h, hw1_ref[...]) + hb1_ref[...]


def kernel(x, cat_vec, params):
    xf = x.astype(jnp.float32)
    px = xf[:, :, 0]
    py = xf[:, :, 1]
    pz = xf[:, :, 2]

    c1x, c1y, c1z, c2x, c2y, c2z = pl.pallas_call(
        _k1_body,
        out_shape=[
            jax.ShapeDtypeStruct((_B, _NP1), jnp.float32),
            jax.ShapeDtypeStruct((_B, _NP1), jnp.float32),
            jax.ShapeDtypeStruct((_B, _NP1), jnp.float32),
            jax.ShapeDtypeStruct((_B, _NP2), jnp.float32),
            jax.ShapeDtypeStruct((_B, _NP2), jnp.float32),
            jax.ShapeDtypeStruct((_B, _NP2), jnp.float32),
        ],
    )(px, py, pz)

    catT = jnp.transpose(cat_vec, (0, 2, 1))
    flat = []
    for name in ('sa1', 'sa2', 'sa3', 'fp3', 'fp2', 'fp1', 'head'):
        for (w, b) in params[name]:
            flat.append(w.astype(jnp.float32))
            flat.append(b.astype(jnp.float32).reshape(1, -1))

    def _bs3(a, b_, c):
        return pl.BlockSpec((1, a, b_) if c else (a, b_),
                            (lambda i: (i, 0, 0)) if c else (lambda i: (0, 0)))

    in_specs = (
        [_bs3(_N, 1, True)] * 3
        + [_bs3(_N, 6, True), _bs3(_N, _CAT, True)]
        + [pl.BlockSpec((1, 1, _NP1), lambda i: (i, 0, 0))] * 3
        + [_bs3(_NP1, 1, True)] * 3
        + [pl.BlockSpec((1, 1, _NP2), lambda i: (i, 0, 0))] * 3
        + [_bs3(_NP2, 1, True)] * 3
        + [_bs3(w.shape[0], w.shape[1], False) for w in flat]
    )

    out = pl.pallas_call(
        _k2_body,
        grid=(_B,),
        in_specs=in_specs,
        out_specs=pl.BlockSpec((1, _N, _NC), lambda i: (i, 0, 0)),
        out_shape=jax.ShapeDtypeStruct((_B, _N, _NC), jnp.float32),
        scratch_shapes=[
            pltpu.VMEM((_N, _NP1), jnp.float32),
            pltpu.VMEM((_N, _NP1), jnp.float32),
            pltpu.VMEM((_N, 128), jnp.float32),
            pltpu.VMEM((_NP1, 128), jnp.float32),
            pltpu.VMEM((_NP1, _NP2), jnp.float32),
            pltpu.VMEM((_NP1, 128), jnp.float32),
        ],
    )(px[..., None], py[..., None], pz[..., None], xf, catT,
      c1x[:, None, :], c1y[:, None, :], c1z[:, None, :],
      c1x[..., None], c1y[..., None], c1z[..., None],
      c2x[:, None, :], c2y[:, None, :], c2z[:, None, :],
      c2x[..., None], c2y[..., None], c2z[..., None],
      *flat)
    return jnp.transpose(out, (0, 2, 1))
